# baseline (speedup 1.0000x reference)
import functools

import jax
import jax.numpy as jnp
from jax import lax
from jax.experimental import pallas as pl
from jax.experimental.pallas import tpu as pltpu
from jax.experimental.pallas import tpu_sc as plsc

_N = 4 * 1 * 224 * 224
_NC = 2
_NS = 16
_CHUNK = _N // (_NC * _NS)
_HALF = _CHUNK // 2
_LANES = 16
_UNROLL = 7
_HVREGS = _HALF // _LANES
_THRESH = 0.2
_INV_N = 1.0 / _N


@jax.jit
def _sc_masked_mean(x_flat):
    mesh = plsc.VectorSubcoreMesh(
        core_axis_name="c", subcore_axis_name="s", num_cores=_NC
    )

    @functools.partial(
        pl.kernel,
        mesh=mesh,
        out_type=jax.ShapeDtypeStruct((_NC, _LANES), jnp.float32),
        scratch_types=[
            pltpu.VMEM((_HALF,), jnp.float32),
            pltpu.VMEM((_HALF,), jnp.float32),
            pltpu.VMEM((1, _LANES), jnp.float32),
            pltpu.VMEM((1,), jnp.int32),
            pltpu.VMEM_SHARED((1, _LANES), jnp.float32),
            pltpu.SemaphoreType.DMA,
            pltpu.SemaphoreType.DMA,
        ],
    )
    def body(x_hbm, out_hbm, x0_v, x1_v, part_v, idx_v, shared, sem0, sem1):
        cid = lax.axis_index("c")
        sid = lax.axis_index("s")
        wid = sid * _NC + cid
        base = wid * _CHUNK

        @pl.when(sid == 0)
        def _():
            part_v[...] = jnp.zeros((1, _LANES), jnp.float32)
            pltpu.sync_copy(part_v, shared)

        cp0 = pltpu.async_copy(x_hbm.at[pl.ds(base, _HALF)], x0_v, sem0)
        cp1 = pltpu.async_copy(
            x_hbm.at[pl.ds(base + _HALF, _HALF)], x1_v, sem1
        )

        zero = jnp.zeros((_LANES,), jnp.float32)

        def half_sum(ref, accs):
            def step(i, accs):
                base_i = i * (_LANES * _UNROLL)
                out = []
                for k in range(_UNROLL):
                    v = jnp.abs(ref[pl.ds(base_i + k * _LANES, _LANES)])
                    out.append(accs[k] + jnp.where(v < _THRESH, v, 0.0))
                return tuple(out)

            return lax.fori_loop(0, _HVREGS // _UNROLL, step, accs)

        cp0.wait()
        accs = half_sum(x0_v, (zero,) * _UNROLL)
        cp1.wait()
        accs = half_sum(x1_v, accs)

        acc = zero
        for k in range(_UNROLL):
            acc = acc + accs[k]

        plsc.subcore_barrier()
        idx_v[...] = jnp.zeros((1,), jnp.int32)
        part_v[0] = acc
        pltpu.sync_copy(part_v, shared.at[idx_v], add=True)
        plsc.subcore_barrier()

        @pl.when(sid == 0)
        def _():
            pltpu.sync_copy(shared, part_v)
            total = part_v[0]
            s = jnp.float32(0.0)
            for j in range(_LANES):
                s = s + total[j]
            part_v[0] = jnp.full((_LANES,), s * _INV_N, jnp.float32)
            pltpu.sync_copy(part_v.at[0], out_hbm.at[cid])

    return body(x_flat)


def kernel(up, left, right):
    del left, right
    out = _sc_masked_mean(up.reshape(-1))
    return out[0, 0] + out[1, 0]


# speedup vs baseline: 1.1989x; 1.1989x over previous
import functools

import jax
import jax.numpy as jnp
from jax import lax
from jax.experimental import pallas as pl
from jax.experimental.pallas import tpu as pltpu
from jax.experimental.pallas import tpu_sc as plsc

_N = 4 * 1 * 224 * 224
_NC = 1
_NS = 16
_CHUNK = _N // (_NC * _NS)
_HALF = _CHUNK // 2
_LANES = 16
_UNROLL = 8
_HVREGS = _HALF // _LANES
_THRESH = 0.2
_INV_N = 1.0 / _N


@jax.jit
def _sc_masked_mean(x_flat):
    mesh = plsc.VectorSubcoreMesh(
        core_axis_name="c", subcore_axis_name="s", num_cores=_NC
    )

    @functools.partial(
        pl.kernel,
        mesh=mesh,
        out_type=jax.ShapeDtypeStruct((_NC, _LANES), jnp.float32),
        scratch_types=[
            pltpu.VMEM((_HALF,), jnp.float32),
            pltpu.VMEM((_HALF,), jnp.float32),
            pltpu.VMEM((1, _LANES), jnp.float32),
            pltpu.VMEM((1,), jnp.int32),
            pltpu.VMEM_SHARED((1, _LANES), jnp.float32),
            pltpu.SemaphoreType.DMA,
            pltpu.SemaphoreType.DMA,
        ],
    )
    def body(x_hbm, out_hbm, x0_v, x1_v, part_v, idx_v, shared, sem0, sem1):
        cid = lax.axis_index("c")
        sid = lax.axis_index("s")
        wid = sid * _NC + cid
        base = wid * _CHUNK

        @pl.when(sid == 0)
        def _():
            part_v[...] = jnp.zeros((1, _LANES), jnp.float32)
            pltpu.sync_copy(part_v, shared)

        cp0 = pltpu.async_copy(x_hbm.at[pl.ds(base, _HALF)], x0_v, sem0)
        cp1 = pltpu.async_copy(
            x_hbm.at[pl.ds(base + _HALF, _HALF)], x1_v, sem1
        )

        zero = jnp.zeros((_LANES,), jnp.float32)

        def half_sum(ref, accs):
            def step(i, accs):
                base_i = i * (_LANES * _UNROLL)
                out = []
                for k in range(_UNROLL):
                    v = jnp.abs(ref[pl.ds(base_i + k * _LANES, _LANES)])
                    out.append(accs[k] + jnp.where(v < _THRESH, v, 0.0))
                return tuple(out)

            return lax.fori_loop(0, _HVREGS // _UNROLL, step, accs)

        cp0.wait()
        accs = half_sum(x0_v, (zero,) * _UNROLL)
        cp1.wait()
        accs = half_sum(x1_v, accs)

        acc = zero
        for k in range(_UNROLL):
            acc = acc + accs[k]

        plsc.subcore_barrier()
        idx_v[...] = jnp.zeros((1,), jnp.int32)
        part_v[0] = acc
        pltpu.sync_copy(part_v, shared.at[idx_v], add=True)
        plsc.subcore_barrier()

        @pl.when(sid == 0)
        def _():
            pltpu.sync_copy(shared, part_v)
            total = part_v[0]
            s = jnp.float32(0.0)
            for j in range(_LANES):
                s = s + total[j]
            part_v[0] = jnp.full((_LANES,), s * _INV_N, jnp.float32)
            pltpu.sync_copy(part_v.at[0], out_hbm.at[cid])

    return body(x_flat)


def kernel(up, left, right):
    del left, right
    out = _sc_masked_mean(up.reshape(-1))
    return out[0, 0]
